# Initial kernel scaffold; baseline (speedup 1.0000x reference)
import jax
import jax.numpy as jnp
from jax.experimental import pallas as pl


def kernel(x, edge_index, W_l, W_r, b_l):
    raise NotImplementedError("write your pallas kernel here")


# speedup vs baseline: 5.3551x; 5.3551x over previous
import functools

import jax
import jax.numpy as jnp
from jax import lax
from jax.experimental import pallas as pl
from jax.experimental.pallas import tpu as pltpu
from jax.experimental.pallas import tpu_sc as plsc

N = 10000
D = 128
DE = 144
E = 320000
NC = 2
NS = 16
NW = NC * NS
EPW = E // NW
C = 80
NCHUNK = EPW // C
NPAD = 10240
RPT = NPAD // NS


def _sc_body(xe_hbm, src_hbm, dst_hbm, sum_hbm, idx_s, idx_d, rows, acc, sem):
  c = lax.axis_index("c")
  s = lax.axis_index("s")
  wid = s * NC + c

  zv = jnp.zeros((16,), jnp.float32)

  def zrow(i, carry):
    for j in range(DE // 16):
      rows[i, pl.ds(j * 16, 16)] = zv
    return carry
  lax.fori_loop(0, C, zrow, 0)

  def zacc(t, carry):
    pltpu.sync_copy(rows, acc.at[pl.ds(s * RPT + t * C, C)])
    return carry
  lax.fori_loop(0, RPT // C, zacc, 0)

  plsc.subcore_barrier()

  base0 = wid * EPW
  def chunk(i, carry):
    base = base0 + i * C
    pltpu.sync_copy(src_hbm.at[pl.ds(base, C)], idx_s)
    pltpu.sync_copy(dst_hbm.at[pl.ds(base, C)], idx_d)
    pltpu.async_copy(xe_hbm.at[idx_s], rows, sem).wait()
    pltpu.sync_copy(rows, acc.at[idx_d], add=True)
    return carry
  lax.fori_loop(0, NCHUNK, chunk, 0)

  plsc.subcore_barrier()

  off = s * RPT
  pltpu.sync_copy(acc.at[pl.ds(off, RPT)], sum_hbm.at[c].at[pl.ds(off, RPT)])


_sc_agg = functools.partial(
    pl.kernel,
    out_type=jax.ShapeDtypeStruct((NC, NPAD, DE), jnp.float32),
    mesh=plsc.VectorSubcoreMesh(core_axis_name="c", subcore_axis_name="s"),
    scratch_types=[
        pltpu.VMEM((C,), jnp.int32),
        pltpu.VMEM((C,), jnp.int32),
        pltpu.VMEM((C, DE), jnp.float32),
        pltpu.VMEM_SHARED((NPAD, DE), jnp.float32),
        pltpu.SemaphoreType.DMA,
    ],
    compiler_params=pltpu.CompilerParams(use_tc_tiling_on_sc=False),
)(_sc_body)


BR = 400


def _dense_body(x_ref, s0_ref, s1_ref, wl_ref, wr_ref, b_ref, o_ref):
  tot = s0_ref[...] + s1_ref[...]
  cnt = jnp.maximum(tot[:, D:D + 1], 1.0)
  mean = tot[:, :D] / cnt
  h = (jnp.dot(mean, wl_ref[...], preferred_element_type=jnp.float32,
               precision=jax.lax.Precision.HIGHEST)
       + jnp.dot(x_ref[...], wr_ref[...], preferred_element_type=jnp.float32,
                 precision=jax.lax.Precision.HIGHEST)
       + b_ref[...])
  o_ref[...] = jnp.maximum(h, 0.0)


def _dense(x, s0, s1, W_l, W_r, b2):
  return pl.pallas_call(
      _dense_body,
      grid=(N // BR,),
      in_specs=[
          pl.BlockSpec((BR, D), lambda i: (i, 0)),
          pl.BlockSpec((BR, DE), lambda i: (i, 0)),
          pl.BlockSpec((BR, DE), lambda i: (i, 0)),
          pl.BlockSpec((D, D), lambda i: (0, 0)),
          pl.BlockSpec((D, D), lambda i: (0, 0)),
          pl.BlockSpec((1, D), lambda i: (0, 0)),
      ],
      out_specs=pl.BlockSpec((BR, D), lambda i: (i, 0)),
      out_shape=jax.ShapeDtypeStruct((N, D), jnp.float32),
  )(x, s0, s1, W_l, W_r, b2)


def kernel(x, edge_index, W_l, W_r, b_l):
  src = edge_index[0].astype(jnp.int32)
  dst = edge_index[1].astype(jnp.int32)
  xe = jnp.concatenate(
      [x, jnp.ones((N, 1), jnp.float32), jnp.zeros((N, DE - D - 1), jnp.float32)],
      axis=1)
  sums = _sc_agg(xe, src, dst)
  return _dense(x, sums[0], sums[1], W_l, W_r, b_l.reshape(1, D))


# speedup vs baseline: 9.5859x; 1.7900x over previous
import functools

import jax
import jax.numpy as jnp
from jax import lax
from jax.experimental import pallas as pl
from jax.experimental.pallas import tpu as pltpu
from jax.experimental.pallas import tpu_sc as plsc

N = 10000
D = 128
DE = 144
E = 320000
NC = 2
NS = 16
NW = NC * NS
EPW = E // NW
C = 40
NCHUNK = EPW // C
NPAD = 10240
RPT = NPAD // NS


NB = 5
NGRP = NCHUNK // NB


def _sc_body(xe_hbm, src_hbm, dst_hbm, sum_hbm, idx_s, idx_d, rows,
             acc, sem_g, sem_s, sem_i):
  c = lax.axis_index("c")
  s = lax.axis_index("s")
  wid = s * NC + c

  zv = jnp.zeros((16,), jnp.float32)

  def zrow(i, carry):
    for j in range(DE // 16):
      rows[0, i, pl.ds(j * 16, 16)] = zv
    return carry
  lax.fori_loop(0, C, zrow, 0)

  def zacc(t, carry):
    pltpu.sync_copy(rows.at[0], acc.at[pl.ds(s * RPT + t * C, C)])
    return carry
  lax.fori_loop(0, RPT // C, zacc, 0)

  row0 = wid * NCHUNK
  pltpu.sync_copy(src_hbm.at[pl.ds(row0, NB)], idx_s.at[0])
  pltpu.sync_copy(dst_hbm.at[pl.ds(row0, NB)], idx_d.at[0])

  plsc.subcore_barrier()

  def one_group(g, p):
    gd = [pltpu.async_copy(xe_hbm.at[idx_s.at[p, b]], rows.at[b], sem_g)
          for b in range(NB)]
    nxt = row0 + (g + 1) * NB
    i1 = pltpu.async_copy(src_hbm.at[pl.ds(nxt, NB)], idx_s.at[1 - p], sem_i)
    i2 = pltpu.async_copy(dst_hbm.at[pl.ds(nxt, NB)], idx_d.at[1 - p], sem_i)
    sd = []
    for b in range(NB):
      gd[b].wait()
      sd.append(pltpu.async_copy(rows.at[b], acc.at[idx_d.at[p, b]],
                                 sem_s, add=True))
    for b in range(NB):
      sd[b].wait()
    i1.wait()
    i2.wait()

  def two_groups(j):
    one_group(2 * j, 0)
    one_group(2 * j + 1, 1)
  pl.loop(0, NGRP // 2)(two_groups)

  plsc.subcore_barrier()

  off = s * RPT
  pltpu.sync_copy(acc.at[pl.ds(off, RPT)], sum_hbm.at[c].at[pl.ds(off, RPT)])


_sc_agg = functools.partial(
    pl.kernel,
    out_type=jax.ShapeDtypeStruct((NC, NPAD, DE), jnp.float32),
    mesh=plsc.VectorSubcoreMesh(core_axis_name="c", subcore_axis_name="s"),
    scratch_types=[
        pltpu.VMEM((2, NB, C), jnp.int32),
        pltpu.VMEM((2, NB, C), jnp.int32),
        pltpu.VMEM((NB, C, DE), jnp.float32),
        pltpu.VMEM_SHARED((NPAD, DE), jnp.float32),
        pltpu.SemaphoreType.DMA,
        pltpu.SemaphoreType.DMA,
        pltpu.SemaphoreType.DMA,
    ],
    compiler_params=pltpu.CompilerParams(use_tc_tiling_on_sc=False),
)(_sc_body)


BR = 400


def _dense_body(x_ref, s0_ref, s1_ref, wl_ref, wr_ref, b_ref, o_ref):
  tot = s0_ref[...] + s1_ref[...]
  cnt = jnp.maximum(tot[:, D:D + 1], 1.0)
  mean = tot[:, :D] / cnt
  h = (jnp.dot(mean, wl_ref[...], preferred_element_type=jnp.float32,
               precision=jax.lax.Precision.HIGHEST)
       + jnp.dot(x_ref[...], wr_ref[...], preferred_element_type=jnp.float32,
                 precision=jax.lax.Precision.HIGHEST)
       + b_ref[...])
  o_ref[...] = jnp.maximum(h, 0.0)


def _dense(x, s0, s1, W_l, W_r, b2):
  return pl.pallas_call(
      _dense_body,
      grid=(N // BR,),
      in_specs=[
          pl.BlockSpec((BR, D), lambda i: (i, 0)),
          pl.BlockSpec((BR, DE), lambda i: (i, 0)),
          pl.BlockSpec((BR, DE), lambda i: (i, 0)),
          pl.BlockSpec((D, D), lambda i: (0, 0)),
          pl.BlockSpec((D, D), lambda i: (0, 0)),
          pl.BlockSpec((1, D), lambda i: (0, 0)),
      ],
      out_specs=pl.BlockSpec((BR, D), lambda i: (i, 0)),
      out_shape=jax.ShapeDtypeStruct((N, D), jnp.float32),
  )(x, s0, s1, W_l, W_r, b2)


def kernel(x, edge_index, W_l, W_r, b_l):
  pad = jnp.zeros((NB * C,), jnp.int32)
  src = jnp.concatenate([edge_index[0].astype(jnp.int32), pad]).reshape(-1, C)
  dst = jnp.concatenate([edge_index[1].astype(jnp.int32), pad]).reshape(-1, C)
  xe = jnp.concatenate(
      [x, jnp.ones((N, 1), jnp.float32), jnp.zeros((N, DE - D - 1), jnp.float32)],
      axis=1)
  sums = _sc_agg(xe, src, dst)
  return _dense(x, sums[0], sums[1], W_l, W_r, b_l.reshape(1, D))
